# baseline (speedup 1.0000x reference)
import jax
import jax.numpy as jnp
from jax import lax
from jax.experimental import pallas as pl
from jax.experimental.pallas import tpu as pltpu
from jax.experimental.pallas import tpu_sc as plsc

_N = 10000
_E = 320000
_D = 128
_R = 6
_NP = 10240
_NC = 2
_NS = 16
_DH = _D // _NC
_DW = _DH + 8
_EP = 327680
_CH = 128
_NJ = _EP // (_NS * _CH)
_RPT = _NP // _NS
_NB = 4
_BLK = 512
_NBLK = _NP // _BLK


def _table_body(x_ref, win_ref, bin_ref, wr_ref, br_ref, src_ref, et_ref,
                h_ref, t0_ref, t1_ref, gidx_ref):
    gidx_ref[...] = et_ref[...] * _NP + src_ref[...]
    xb = x_ref[...]
    h = jnp.maximum(
        lax.dot_general(xb, win_ref[...], (((1,), (1,)), ((), ())),
                        preferred_element_type=jnp.float32) + bin_ref[...],
        0.0)
    h_ref[...] = h
    ones = jnp.ones((_BLK, _DW - _DH), jnp.float32)
    for r in range(_R):
        row = lax.dot_general(h, wr_ref[r], (((1,), (1,)), ((), ())),
                              preferred_element_type=jnp.float32) + br_ref[r]
        t0_ref[r] = jnp.concatenate([row[:, :_DH], ones], axis=1)
        t1_ref[r] = jnp.concatenate([row[:, _DH:], ones], axis=1)


def _table_call(x_p, W_in, b_in, Wr, br, src2d, et2d):
    return pl.pallas_call(
        _table_body,
        grid=(_NBLK,),
        in_specs=[
            pl.BlockSpec((_BLK, _D), lambda i: (i, 0)),
            pl.BlockSpec((_D, _D), lambda i: (0, 0)),
            pl.BlockSpec((1, _D), lambda i: (0, 0)),
            pl.BlockSpec((_R, _D, _D), lambda i: (0, 0, 0)),
            pl.BlockSpec((_R, 1, _D), lambda i: (0, 0, 0)),
            pl.BlockSpec((_EP // _D // _NBLK, _D), lambda i: (i, 0)),
            pl.BlockSpec((_EP // _D // _NBLK, _D), lambda i: (i, 0)),
        ],
        out_specs=[
            pl.BlockSpec((_BLK, _D), lambda i: (i, 0)),
            pl.BlockSpec((_R, _BLK, _DW), lambda i: (0, i, 0)),
            pl.BlockSpec((_R, _BLK, _DW), lambda i: (0, i, 0)),
            pl.BlockSpec((_EP // _D // _NBLK, _D), lambda i: (i, 0)),
        ],
        out_shape=[
            jax.ShapeDtypeStruct((_NP, _D), jnp.float32),
            jax.ShapeDtypeStruct((_R, _NP, _DW), jnp.float32),
            jax.ShapeDtypeStruct((_R, _NP, _DW), jnp.float32),
            jax.ShapeDtypeStruct((_EP // _D, _D), jnp.int32),
        ],
    )(x_p, W_in, b_in, Wr, br, src2d, et2d)


def _sc_body(t0_hbm, t1_hbm, gidx_hbm, dst_hbm, zrow_hbm, msgp_hbm,
             idx_v, dst_v, *bufs_and_sems):
    rows = bufs_and_sems[:_NB]
    msg_sh = bufs_and_sems[_NB]
    gsem = bufs_and_sems[_NB + 1:2 * _NB + 1]
    ssem = bufs_and_sems[2 * _NB + 1:]
    c = lax.axis_index("c")
    s = lax.axis_index("s")

    for k in range(_RPT // _CH):
        pltpu.sync_copy(zrow_hbm, msg_sh.at[pl.ds(s * _RPT + k * _CH, _CH)])
    pltpu.sync_copy(gidx_hbm.at[s], idx_v)
    pltpu.sync_copy(dst_hbm.at[s], dst_v)
    plsc.subcore_barrier()

    def start(j, buf, sem):
        @pl.when(c == 0)
        def _():
            pltpu.async_copy(t0_hbm.at[idx_v.at[j]], buf, sem)

        @pl.when(c == 1)
        def _():
            pltpu.async_copy(t1_hbm.at[idx_v.at[j]], buf, sem)

    def wait_g(buf, sem):
        pltpu.make_async_copy(t0_hbm.at[pl.ds(0, _CH)], buf, sem).wait()

    def wait_s(buf, sem):
        pltpu.make_async_copy(buf, msg_sh.at[pl.ds(0, _CH)], sem).wait()

    for k in range(2):
        start(k, rows[k], gsem[k])

    def step(i, carry):
        for k in range(_NB):
            jj = i * _NB + k

            @pl.when(jj + 2 < _NJ)
            def _():
                kn = (k + 2) % _NB

                @pl.when(jj >= 2)
                def _():
                    wait_s(rows[kn], ssem[kn])

                start(jj + 2, rows[kn], gsem[kn])

            wait_g(rows[k], gsem[k])
            pltpu.async_copy(rows[k], msg_sh.at[dst_v.at[jj]], ssem[k],
                             add=True)

        return carry

    lax.fori_loop(0, _NJ // _NB, step, 0)

    for k in range(_NB):
        wait_s(rows[k], ssem[k])

    plsc.subcore_barrier()
    pltpu.sync_copy(msg_sh.at[pl.ds(s * _RPT, _RPT)],
                    msgp_hbm.at[c, pl.ds(s * _RPT, _RPT)])


def _sc_call(t0, t1, gidx, dst, zrow):
    fn = pl.kernel(
        _sc_body,
        out_type=jax.ShapeDtypeStruct((_NC, _NP, _DW), jnp.float32),
        mesh=plsc.VectorSubcoreMesh(core_axis_name="c", subcore_axis_name="s"),
        compiler_params=pltpu.CompilerParams(use_tc_tiling_on_sc=False),
        scratch_types=[
            pltpu.VMEM((_NJ, _CH), jnp.int32),
            pltpu.VMEM((_NJ, _CH), jnp.int32),
        ] + [pltpu.VMEM((_CH, _DW), jnp.float32) for _ in range(_NB)] + [
            pltpu.VMEM_SHARED((_NP, _DW), jnp.float32),
        ] + [pltpu.SemaphoreType.DMA for _ in range(2 * _NB)],
    )
    return fn(t0, t1, gidx, dst, zrow)


def _epi_body(h_ref, msgp_ref, w1_ref, b1_ref, w2_ref, b2_ref,
              wp_ref, emb_ref, pred_ref):
    msg = jnp.concatenate([msgp_ref[0, :, :_DH], msgp_ref[1, :, :_DH]], axis=1)
    cnt = msgp_ref[0, :, _DH:_DH + 1]
    t = h_ref[...] + msg / jnp.maximum(cnt, 1.0)
    z = jnp.maximum(
        lax.dot_general(t, w1_ref[...], (((1,), (1,)), ((), ())),
                        preferred_element_type=jnp.float32) + b1_ref[...],
        0.0)
    emb = lax.dot_general(z, w2_ref[...], (((1,), (1,)), ((), ())),
                          preferred_element_type=jnp.float32) + b2_ref[...]
    emb_ref[...] = emb
    pred_ref[...] = lax.dot_general(emb, wp_ref[...], (((1,), (1,)), ((), ())),
                                    preferred_element_type=jnp.float32)


def _epi_call(h, msgp, W1, b1, W2, b2, Wp):
    return pl.pallas_call(
        _epi_body,
        grid=(_NBLK,),
        in_specs=[
            pl.BlockSpec((_BLK, _D), lambda i: (i, 0)),
            pl.BlockSpec((_NC, _BLK, _DW), lambda i: (0, i, 0)),
            pl.BlockSpec((_D, _D), lambda i: (0, 0)),
            pl.BlockSpec((1, _D), lambda i: (0, 0)),
            pl.BlockSpec((_D, _D), lambda i: (0, 0)),
            pl.BlockSpec((1, _D), lambda i: (0, 0)),
            pl.BlockSpec((1, _D), lambda i: (0, 0)),
        ],
        out_specs=[
            pl.BlockSpec((_BLK, _D), lambda i: (i, 0)),
            pl.BlockSpec((_BLK, 1), lambda i: (i, 0)),
        ],
        out_shape=[
            jax.ShapeDtypeStruct((_NP, _D), jnp.float32),
            jax.ShapeDtypeStruct((_NP, 1), jnp.float32),
        ],
    )(h, msgp, W1, b1, W2, b2, Wp)


@jax.jit
def kernel(x, edge_index, edge_type, W_in, b_in, Wr, br, W1, b1, W2, b2, Wp, bp):
    x_p = jnp.pad(x, ((0, _NP - _N), (0, 0)))
    pad = _EP - _E
    src_p = jnp.pad(edge_index[0], (0, pad))
    et_p = jnp.pad(edge_type, (0, pad))
    dst_p = jnp.pad(edge_index[1], (0, pad), constant_values=_N)
    h, T0, T1, gidx2d = _table_call(x_p, W_in, b_in.reshape(1, _D), Wr,
                                    br.reshape(_R, 1, _D),
                                    src_p.reshape(_EP // _D, _D),
                                    et_p.reshape(_EP // _D, _D))
    t0 = T0.reshape(_R * _NP, _DW)
    t1 = T1.reshape(_R * _NP, _DW)
    gidx = gidx2d.reshape(_NS, _NJ, _CH)
    dst = dst_p.reshape(_NS, _NJ, _CH)

    zrow = jnp.zeros((_CH, _DW), jnp.float32)
    msgp = _sc_call(t0, t1, gidx, dst, zrow)

    emb_p, pred_p = _epi_call(h, msgp, W1, b1.reshape(1, _D), W2,
                              b2.reshape(1, _D), Wp)
    return emb_p[:_N], pred_p[:_N] + bp


# speedup vs baseline: 1.4577x; 1.4577x over previous
import jax
import jax.numpy as jnp
from jax import lax
from jax.experimental import pallas as pl
from jax.experimental.pallas import tpu as pltpu
from jax.experimental.pallas import tpu_sc as plsc

_N = 10000
_E = 320000
_D = 128
_R = 6
_NP = 10240
_NC = 2
_NS = 16
_DH = _D // _NC
_DW = _DH + 16
_CH = 112
_NJ = 180
_EP = _NS * _NJ * _CH
_RPT = _NP // _NS
_NB = 4
_BLK = 512
_NBLK = _NP // _BLK


def _gidx_body(src_ref, et_ref, gidx_ref):
    gidx_ref[...] = et_ref[...] * _NP + src_ref[...]


def _gidx_call(src2d, et2d):
    rows = _EP // _D
    return pl.pallas_call(
        _gidx_body,
        grid=(5,),
        in_specs=[
            pl.BlockSpec((rows // 5, _D), lambda i: (i, 0)),
            pl.BlockSpec((rows // 5, _D), lambda i: (i, 0)),
        ],
        out_specs=pl.BlockSpec((rows // 5, _D), lambda i: (i, 0)),
        out_shape=jax.ShapeDtypeStruct((rows, _D), jnp.int32),
    )(src2d, et2d)


def _table_body(x_ref, win_ref, bin_ref, wr_ref, br_ref,
                h_ref, t0_ref, t1_ref):
    xb = x_ref[...]
    h = jnp.maximum(
        lax.dot_general(xb, win_ref[...], (((1,), (1,)), ((), ())),
                        preferred_element_type=jnp.float32) + bin_ref[...],
        0.0)
    h_ref[...] = h
    ones = jnp.ones((_BLK, _DW - _DH), jnp.float32)
    for r in range(_R):
        row = lax.dot_general(h, wr_ref[r], (((1,), (1,)), ((), ())),
                              preferred_element_type=jnp.float32) + br_ref[r]
        t0_ref[r] = jnp.concatenate([row[:, :_DH], ones], axis=1)
        t1_ref[r] = jnp.concatenate([row[:, _DH:], ones], axis=1)


def _table_call(x_p, W_in, b_in, Wr, br):
    return pl.pallas_call(
        _table_body,
        grid=(_NBLK,),
        in_specs=[
            pl.BlockSpec((_BLK, _D), lambda i: (i, 0)),
            pl.BlockSpec((_D, _D), lambda i: (0, 0)),
            pl.BlockSpec((1, _D), lambda i: (0, 0)),
            pl.BlockSpec((_R, _D, _D), lambda i: (0, 0, 0)),
            pl.BlockSpec((_R, 1, _D), lambda i: (0, 0, 0)),
        ],
        out_specs=[
            pl.BlockSpec((_BLK, _D), lambda i: (i, 0)),
            pl.BlockSpec((_R, _BLK, _DW), lambda i: (0, i, 0)),
            pl.BlockSpec((_R, _BLK, _DW), lambda i: (0, i, 0)),
        ],
        out_shape=[
            jax.ShapeDtypeStruct((_NP, _D), jnp.float32),
            jax.ShapeDtypeStruct((_R, _NP, _DW), jnp.float32),
            jax.ShapeDtypeStruct((_R, _NP, _DW), jnp.float32),
        ],
    )(x_p, W_in, b_in, Wr, br)


def _sc_body(t0_hbm, t1_hbm, gidx_hbm, dst_hbm, zrow_hbm, msgp_hbm,
             idx_v, dst_v, *bufs_and_sems):
    rows = bufs_and_sems[:_NB]
    msg_sh = bufs_and_sems[_NB]
    gsem = bufs_and_sems[_NB + 1:2 * _NB + 1]
    ssem = bufs_and_sems[2 * _NB + 1:]
    c = lax.axis_index("c")
    s = lax.axis_index("s")

    pltpu.sync_copy(zrow_hbm, msg_sh.at[pl.ds(s * _RPT, _RPT)])
    pltpu.sync_copy(gidx_hbm.at[s], idx_v)
    pltpu.sync_copy(dst_hbm.at[s], dst_v)
    plsc.subcore_barrier()

    def start(j, buf, sem):
        @pl.when(c == 0)
        def _():
            pltpu.async_copy(t0_hbm.at[idx_v.at[j]], buf, sem)

        @pl.when(c == 1)
        def _():
            pltpu.async_copy(t1_hbm.at[idx_v.at[j]], buf, sem)

    def wait_g(buf, sem):
        pltpu.make_async_copy(t0_hbm.at[pl.ds(0, _CH)], buf, sem).wait()

    def wait_s(buf, sem):
        pltpu.make_async_copy(buf, msg_sh.at[pl.ds(0, _CH)], sem).wait()

    for k in range(2):
        start(k, rows[k], gsem[k])

    def step(i, carry):
        for k in range(_NB):
            jj = i * _NB + k

            @pl.when(jj + 2 < _NJ)
            def _():
                kn = (k + 2) % _NB

                @pl.when(jj >= 2)
                def _():
                    wait_s(rows[kn], ssem[kn])

                start(jj + 2, rows[kn], gsem[kn])

            wait_g(rows[k], gsem[k])
            pltpu.async_copy(rows[k], msg_sh.at[dst_v.at[jj]], ssem[k],
                             add=True)

        return carry

    lax.fori_loop(0, _NJ // _NB, step, 0)

    for k in range(_NB):
        wait_s(rows[k], ssem[k])

    plsc.subcore_barrier()
    pltpu.sync_copy(msg_sh.at[pl.ds(s * _RPT, _RPT)],
                    msgp_hbm.at[c, pl.ds(s * _RPT, _RPT)])


def _sc_call(t0, t1, gidx, dst, zrow):
    fn = pl.kernel(
        _sc_body,
        out_type=jax.ShapeDtypeStruct((_NC, _NP, _DW), jnp.float32),
        mesh=plsc.VectorSubcoreMesh(core_axis_name="c", subcore_axis_name="s"),
        compiler_params=pltpu.CompilerParams(use_tc_tiling_on_sc=False),
        scratch_types=[
            pltpu.VMEM((_NJ, _CH), jnp.int32),
            pltpu.VMEM((_NJ, _CH), jnp.int32),
        ] + [pltpu.VMEM((_CH, _DW), jnp.float32) for _ in range(_NB)] + [
            pltpu.VMEM_SHARED((_NP, _DW), jnp.float32),
        ] + [pltpu.SemaphoreType.DMA for _ in range(2 * _NB)],
    )
    return fn(t0, t1, gidx, dst, zrow)


def _epi_body(h_ref, msgp_ref, w1_ref, b1_ref, w2_ref, b2_ref,
              wp_ref, emb_ref, pred_ref):
    msg = jnp.concatenate([msgp_ref[0, :, :_DH], msgp_ref[1, :, :_DH]], axis=1)
    cnt = msgp_ref[0, :, _DH:_DH + 1]
    t = h_ref[...] + msg / jnp.maximum(cnt, 1.0)
    z = jnp.maximum(
        lax.dot_general(t, w1_ref[...], (((1,), (1,)), ((), ())),
                        preferred_element_type=jnp.float32) + b1_ref[...],
        0.0)
    emb = lax.dot_general(z, w2_ref[...], (((1,), (1,)), ((), ())),
                          preferred_element_type=jnp.float32) + b2_ref[...]
    emb_ref[...] = emb
    pred_ref[...] = lax.dot_general(emb, wp_ref[...], (((1,), (1,)), ((), ())),
                                    preferred_element_type=jnp.float32)


def _epi_call(h, msgp, W1, b1, W2, b2, Wp):
    return pl.pallas_call(
        _epi_body,
        grid=(_NBLK,),
        in_specs=[
            pl.BlockSpec((_BLK, _D), lambda i: (i, 0)),
            pl.BlockSpec((_NC, _BLK, _DW), lambda i: (0, i, 0)),
            pl.BlockSpec((_D, _D), lambda i: (0, 0)),
            pl.BlockSpec((1, _D), lambda i: (0, 0)),
            pl.BlockSpec((_D, _D), lambda i: (0, 0)),
            pl.BlockSpec((1, _D), lambda i: (0, 0)),
            pl.BlockSpec((1, _D), lambda i: (0, 0)),
        ],
        out_specs=[
            pl.BlockSpec((_BLK, _D), lambda i: (i, 0)),
            pl.BlockSpec((_BLK, 1), lambda i: (i, 0)),
        ],
        out_shape=[
            jax.ShapeDtypeStruct((_NP, _D), jnp.float32),
            jax.ShapeDtypeStruct((_NP, 1), jnp.float32),
        ],
    )(h, msgp, W1, b1, W2, b2, Wp)


@jax.jit
def kernel(x, edge_index, edge_type, W_in, b_in, Wr, br, W1, b1, W2, b2, Wp, bp):
    x_p = jnp.pad(x, ((0, _NP - _N), (0, 0)))
    pad = _EP - _E
    src_p = jnp.pad(edge_index[0], (0, pad))
    et_p = jnp.pad(edge_type, (0, pad))
    dst_p = jnp.pad(edge_index[1], (0, pad), constant_values=_N)
    h, T0, T1 = _table_call(x_p, W_in, b_in.reshape(1, _D), Wr,
                            br.reshape(_R, 1, _D))
    gidx2d = _gidx_call(src_p.reshape(_EP // _D, _D),
                        et_p.reshape(_EP // _D, _D))
    t0 = T0.reshape(_R * _NP, _DW)
    t1 = T1.reshape(_R * _NP, _DW)
    gidx = gidx2d.reshape(_NS, _NJ, _CH)
    dst = dst_p.reshape(_NS, _NJ, _CH)

    zrow = jnp.zeros((_RPT, _DW), jnp.float32)
    msgp = _sc_call(t0, t1, gidx, dst, zrow)

    emb_p, pred_p = _epi_call(h, msgp, W1, b1.reshape(1, _D), W2,
                              b2.reshape(1, _D), Wp)
    return emb_p[:_N], pred_p[:_N] + bp


# speedup vs baseline: 1.4924x; 1.0238x over previous
import jax
import jax.numpy as jnp
from jax import lax
from jax.experimental import pallas as pl
from jax.experimental.pallas import tpu as pltpu
from jax.experimental.pallas import tpu_sc as plsc

_N = 10000
_E = 320000
_D = 128
_R = 6
_NP = 10240
_NC = 2
_NS = 16
_DH = _D // _NC
_DW = _DH + 16
_CH = 80
_NJ = 252
_EP = _NS * _NJ * _CH
_RPT = _NP // _NS
_NB = 6
_BLK = 512
_NBLK = _NP // _BLK


def _gidx_body(src_ref, et_ref, gidx_ref):
    gidx_ref[...] = et_ref[...] * _NP + src_ref[...]


def _gidx_call(src2d, et2d):
    rows = _EP // _D
    return pl.pallas_call(
        _gidx_body,
        grid=(5,),
        in_specs=[
            pl.BlockSpec((rows // 5, _D), lambda i: (i, 0)),
            pl.BlockSpec((rows // 5, _D), lambda i: (i, 0)),
        ],
        out_specs=pl.BlockSpec((rows // 5, _D), lambda i: (i, 0)),
        out_shape=jax.ShapeDtypeStruct((rows, _D), jnp.int32),
    )(src2d, et2d)


def _table_body(x_ref, win_ref, bin_ref, wr_ref, br_ref,
                h_ref, t0_ref, t1_ref):
    xb = x_ref[...]
    h = jnp.maximum(
        lax.dot_general(xb, win_ref[...], (((1,), (1,)), ((), ())),
                        preferred_element_type=jnp.float32) + bin_ref[...],
        0.0)
    h_ref[...] = h
    ones = jnp.ones((_BLK, _DW - _DH), jnp.float32)
    for r in range(_R):
        row = lax.dot_general(h, wr_ref[r], (((1,), (1,)), ((), ())),
                              preferred_element_type=jnp.float32) + br_ref[r]
        t0_ref[r] = jnp.concatenate([row[:, :_DH], ones], axis=1)
        t1_ref[r] = jnp.concatenate([row[:, _DH:], ones], axis=1)


def _table_call(x_p, W_in, b_in, Wr, br):
    return pl.pallas_call(
        _table_body,
        grid=(_NBLK,),
        in_specs=[
            pl.BlockSpec((_BLK, _D), lambda i: (i, 0)),
            pl.BlockSpec((_D, _D), lambda i: (0, 0)),
            pl.BlockSpec((1, _D), lambda i: (0, 0)),
            pl.BlockSpec((_R, _D, _D), lambda i: (0, 0, 0)),
            pl.BlockSpec((_R, 1, _D), lambda i: (0, 0, 0)),
        ],
        out_specs=[
            pl.BlockSpec((_BLK, _D), lambda i: (i, 0)),
            pl.BlockSpec((_R, _BLK, _DW), lambda i: (0, i, 0)),
            pl.BlockSpec((_R, _BLK, _DW), lambda i: (0, i, 0)),
        ],
        out_shape=[
            jax.ShapeDtypeStruct((_NP, _D), jnp.float32),
            jax.ShapeDtypeStruct((_R, _NP, _DW), jnp.float32),
            jax.ShapeDtypeStruct((_R, _NP, _DW), jnp.float32),
        ],
    )(x_p, W_in, b_in, Wr, br)


def _sc_body(t0_hbm, t1_hbm, gidx_hbm, dst_hbm, zrow_hbm, msgp_hbm,
             idx_v, dst_v, *bufs_and_sems):
    rows = bufs_and_sems[:_NB]
    msg_sh = bufs_and_sems[_NB]
    gsem = bufs_and_sems[_NB + 1:2 * _NB + 1]
    ssem = bufs_and_sems[2 * _NB + 1:]
    c = lax.axis_index("c")
    s = lax.axis_index("s")

    pltpu.sync_copy(zrow_hbm, msg_sh.at[pl.ds(s * _RPT, _RPT)])
    pltpu.sync_copy(gidx_hbm.at[s], idx_v)
    pltpu.sync_copy(dst_hbm.at[s], dst_v)
    plsc.subcore_barrier()

    def start(j, buf, sem):
        @pl.when(c == 0)
        def _():
            pltpu.async_copy(t0_hbm.at[idx_v.at[j]], buf, sem)

        @pl.when(c == 1)
        def _():
            pltpu.async_copy(t1_hbm.at[idx_v.at[j]], buf, sem)

    def wait_g(buf, sem):
        pltpu.make_async_copy(t0_hbm.at[pl.ds(0, _CH)], buf, sem).wait()

    def wait_s(buf, sem):
        pltpu.make_async_copy(buf, msg_sh.at[pl.ds(0, _CH)], sem).wait()

    for k in range(4):
        start(k, rows[k], gsem[k])

    def step(i, carry):
        for k in range(_NB):
            jj = i * _NB + k

            @pl.when(jj + 4 < _NJ)
            def _():
                kn = (k + 4) % _NB

                @pl.when(jj >= 2)
                def _():
                    wait_s(rows[kn], ssem[kn])

                start(jj + 4, rows[kn], gsem[kn])

            wait_g(rows[k], gsem[k])
            pltpu.async_copy(rows[k], msg_sh.at[dst_v.at[jj]], ssem[k],
                             add=True)

        return carry

    lax.fori_loop(0, _NJ // _NB, step, 0)

    for k in range(_NB):
        wait_s(rows[k], ssem[k])

    plsc.subcore_barrier()
    pltpu.sync_copy(msg_sh.at[pl.ds(s * _RPT, _RPT)],
                    msgp_hbm.at[c, pl.ds(s * _RPT, _RPT)])


def _sc_call(t0, t1, gidx, dst, zrow):
    fn = pl.kernel(
        _sc_body,
        out_type=jax.ShapeDtypeStruct((_NC, _NP, _DW), jnp.float32),
        mesh=plsc.VectorSubcoreMesh(core_axis_name="c", subcore_axis_name="s"),
        compiler_params=pltpu.CompilerParams(use_tc_tiling_on_sc=False),
        scratch_types=[
            pltpu.VMEM((_NJ, _CH), jnp.int32),
            pltpu.VMEM((_NJ, _CH), jnp.int32),
        ] + [pltpu.VMEM((_CH, _DW), jnp.float32) for _ in range(_NB)] + [
            pltpu.VMEM_SHARED((_NP, _DW), jnp.float32),
        ] + [pltpu.SemaphoreType.DMA for _ in range(2 * _NB)],
    )
    return fn(t0, t1, gidx, dst, zrow)


def _epi_body(h_ref, msgp_ref, w1_ref, b1_ref, w2_ref, b2_ref,
              wp_ref, emb_ref, pred_ref):
    msg = jnp.concatenate([msgp_ref[0, :, :_DH], msgp_ref[1, :, :_DH]], axis=1)
    cnt = msgp_ref[0, :, _DH:_DH + 1]
    t = h_ref[...] + msg / jnp.maximum(cnt, 1.0)
    z = jnp.maximum(
        lax.dot_general(t, w1_ref[...], (((1,), (1,)), ((), ())),
                        preferred_element_type=jnp.float32) + b1_ref[...],
        0.0)
    emb = lax.dot_general(z, w2_ref[...], (((1,), (1,)), ((), ())),
                          preferred_element_type=jnp.float32) + b2_ref[...]
    emb_ref[...] = emb
    pred_ref[...] = lax.dot_general(emb, wp_ref[...], (((1,), (1,)), ((), ())),
                                    preferred_element_type=jnp.float32)


def _epi_call(h, msgp, W1, b1, W2, b2, Wp):
    return pl.pallas_call(
        _epi_body,
        grid=(_NBLK,),
        in_specs=[
            pl.BlockSpec((_BLK, _D), lambda i: (i, 0)),
            pl.BlockSpec((_NC, _BLK, _DW), lambda i: (0, i, 0)),
            pl.BlockSpec((_D, _D), lambda i: (0, 0)),
            pl.BlockSpec((1, _D), lambda i: (0, 0)),
            pl.BlockSpec((_D, _D), lambda i: (0, 0)),
            pl.BlockSpec((1, _D), lambda i: (0, 0)),
            pl.BlockSpec((1, _D), lambda i: (0, 0)),
        ],
        out_specs=[
            pl.BlockSpec((_BLK, _D), lambda i: (i, 0)),
            pl.BlockSpec((_BLK, 1), lambda i: (i, 0)),
        ],
        out_shape=[
            jax.ShapeDtypeStruct((_NP, _D), jnp.float32),
            jax.ShapeDtypeStruct((_NP, 1), jnp.float32),
        ],
    )(h, msgp, W1, b1, W2, b2, Wp)


@jax.jit
def kernel(x, edge_index, edge_type, W_in, b_in, Wr, br, W1, b1, W2, b2, Wp, bp):
    x_p = jnp.pad(x, ((0, _NP - _N), (0, 0)))
    pad = _EP - _E
    src_p = jnp.pad(edge_index[0], (0, pad))
    et_p = jnp.pad(edge_type, (0, pad))
    dst_p = jnp.pad(edge_index[1], (0, pad), constant_values=_N)
    h, T0, T1 = _table_call(x_p, W_in, b_in.reshape(1, _D), Wr,
                            br.reshape(_R, 1, _D))
    gidx2d = _gidx_call(src_p.reshape(_EP // _D, _D),
                        et_p.reshape(_EP // _D, _D))
    t0 = T0.reshape(_R * _NP, _DW)
    t1 = T1.reshape(_R * _NP, _DW)
    gidx = gidx2d.reshape(_NS, _NJ, _CH)
    dst = dst_p.reshape(_NS, _NJ, _CH)

    zrow = jnp.zeros((_RPT, _DW), jnp.float32)
    msgp = _sc_call(t0, t1, gidx, dst, zrow)

    emb_p, pred_p = _epi_call(h, msgp, W1, b1.reshape(1, _D), W2,
                              b2.reshape(1, _D), Wp)
    return emb_p[:_N], pred_p[:_N] + bp


# speedup vs baseline: 1.6595x; 1.1119x over previous
import jax
import jax.numpy as jnp
from jax import lax
from jax.experimental import pallas as pl
from jax.experimental.pallas import tpu as pltpu
from jax.experimental.pallas import tpu_sc as plsc

_N = 10000
_E = 320000
_D = 128
_R = 6
_NP = 10240
_NC = 2
_NS = 16
_DH = _D // _NC
_DW = _DH + 16
_CH = 80
_NJ = 252
_EP = _NS * _NJ * _CH
_RPT = _NP // _NS
_NB = 6
_BLK = 512
_NBLK = _NP // _BLK


_EPP = _NP * 32


def _table_body(x_ref, win_ref, bin_ref, wr_ref, br_ref, src_ref, et_ref,
                h_ref, t0_ref, t1_ref, gidx_ref):
    gidx_ref[...] = et_ref[...] * _NP + src_ref[...]
    xb = x_ref[...]
    h = jnp.maximum(
        lax.dot_general(xb, win_ref[...], (((1,), (1,)), ((), ())),
                        preferred_element_type=jnp.float32) + bin_ref[...],
        0.0)
    h_ref[...] = h
    ones = jnp.ones((_BLK, _DW - _DH), jnp.float32)
    for r in range(_R):
        row = lax.dot_general(h, wr_ref[r], (((1,), (1,)), ((), ())),
                              preferred_element_type=jnp.float32) + br_ref[r]
        t0_ref[r] = jnp.concatenate([row[:, :_DH], ones], axis=1)
        t1_ref[r] = jnp.concatenate([row[:, _DH:], ones], axis=1)


def _table_call(x_p, W_in, b_in, Wr, br, src2d, et2d):
    erows = _EPP // _D // _NBLK
    return pl.pallas_call(
        _table_body,
        grid=(_NBLK,),
        in_specs=[
            pl.BlockSpec((_BLK, _D), lambda i: (i, 0)),
            pl.BlockSpec((_D, _D), lambda i: (0, 0)),
            pl.BlockSpec((1, _D), lambda i: (0, 0)),
            pl.BlockSpec((_R, _D, _D), lambda i: (0, 0, 0)),
            pl.BlockSpec((_R, 1, _D), lambda i: (0, 0, 0)),
            pl.BlockSpec((erows, _D), lambda i: (i, 0)),
            pl.BlockSpec((erows, _D), lambda i: (i, 0)),
        ],
        out_specs=[
            pl.BlockSpec((_BLK, _D), lambda i: (i, 0)),
            pl.BlockSpec((_R, _BLK, _DW), lambda i: (0, i, 0)),
            pl.BlockSpec((_R, _BLK, _DW), lambda i: (0, i, 0)),
            pl.BlockSpec((erows, _D), lambda i: (i, 0)),
        ],
        out_shape=[
            jax.ShapeDtypeStruct((_NP, _D), jnp.float32),
            jax.ShapeDtypeStruct((_R, _NP, _DW), jnp.float32),
            jax.ShapeDtypeStruct((_R, _NP, _DW), jnp.float32),
            jax.ShapeDtypeStruct((_EPP // _D, _D), jnp.int32),
        ],
    )(x_p, W_in, b_in, Wr, br, src2d, et2d)


def _sc_body(t0_hbm, t1_hbm, gidx_hbm, dst_hbm, zrow_hbm, msgp_hbm,
             idx_v, dst_v, *bufs_and_sems):
    rows = bufs_and_sems[:_NB]
    msg_sh = bufs_and_sems[_NB]
    gsem = bufs_and_sems[_NB + 1:2 * _NB + 1]
    ssem = bufs_and_sems[2 * _NB + 1:]
    c = lax.axis_index("c")
    s = lax.axis_index("s")

    pltpu.sync_copy(zrow_hbm, msg_sh.at[pl.ds(s * _RPT, _RPT)])
    pltpu.sync_copy(gidx_hbm.at[s], idx_v)
    pltpu.sync_copy(dst_hbm.at[s], dst_v)
    plsc.subcore_barrier()

    def start(j, buf, sem):
        @pl.when(c == 0)
        def _():
            pltpu.async_copy(t0_hbm.at[idx_v.at[j]], buf, sem)

        @pl.when(c == 1)
        def _():
            pltpu.async_copy(t1_hbm.at[idx_v.at[j]], buf, sem)

    def wait_g(buf, sem):
        pltpu.make_async_copy(t0_hbm.at[pl.ds(0, _CH)], buf, sem).wait()

    def wait_s(buf, sem):
        pltpu.make_async_copy(buf, msg_sh.at[pl.ds(0, _CH)], sem).wait()

    for k in range(4):
        start(k, rows[k], gsem[k])

    def step(i, carry):
        for k in range(_NB):
            jj = i * _NB + k

            @pl.when(jj + 4 < _NJ)
            def _():
                kn = (k + 4) % _NB

                @pl.when(jj >= 2)
                def _():
                    wait_s(rows[kn], ssem[kn])

                start(jj + 4, rows[kn], gsem[kn])

            wait_g(rows[k], gsem[k])
            pltpu.async_copy(rows[k], msg_sh.at[dst_v.at[jj]], ssem[k],
                             add=True)

        return carry

    lax.fori_loop(0, _NJ // _NB, step, 0)

    for k in range(_NB):
        wait_s(rows[k], ssem[k])

    plsc.subcore_barrier()
    pltpu.sync_copy(msg_sh.at[pl.ds(s * _RPT, _RPT)],
                    msgp_hbm.at[c, pl.ds(s * _RPT, _RPT)])


def _sc_call(t0, t1, gidx, dst, zrow):
    fn = pl.kernel(
        _sc_body,
        out_type=jax.ShapeDtypeStruct((_NC, _NP, _DW), jnp.float32),
        mesh=plsc.VectorSubcoreMesh(core_axis_name="c", subcore_axis_name="s"),
        compiler_params=pltpu.CompilerParams(use_tc_tiling_on_sc=False),
        scratch_types=[
            pltpu.VMEM((_NJ, _CH), jnp.int32),
            pltpu.VMEM((_NJ, _CH), jnp.int32),
        ] + [pltpu.VMEM((_CH, _DW), jnp.float32) for _ in range(_NB)] + [
            pltpu.VMEM_SHARED((_NP, _DW), jnp.float32),
        ] + [pltpu.SemaphoreType.DMA for _ in range(2 * _NB)],
    )
    return fn(t0, t1, gidx, dst, zrow)


def _epi_body(h_ref, msgp_ref, w1_ref, b1_ref, w2_ref, b2_ref,
              wp_ref, emb_ref, pred_ref):
    msg = jnp.concatenate([msgp_ref[0, :, :_DH], msgp_ref[1, :, :_DH]], axis=1)
    cnt = msgp_ref[0, :, _DH:_DH + 1]
    t = h_ref[...] + msg / jnp.maximum(cnt, 1.0)
    z = jnp.maximum(
        lax.dot_general(t, w1_ref[...], (((1,), (1,)), ((), ())),
                        preferred_element_type=jnp.float32) + b1_ref[...],
        0.0)
    emb = lax.dot_general(z, w2_ref[...], (((1,), (1,)), ((), ())),
                          preferred_element_type=jnp.float32) + b2_ref[...]
    emb_ref[...] = emb
    pred_ref[...] = lax.dot_general(emb, wp_ref[...], (((1,), (1,)), ((), ())),
                                    preferred_element_type=jnp.float32)


def _epi_call(h, msgp, W1, b1, W2, b2, Wp):
    return pl.pallas_call(
        _epi_body,
        grid=(_NBLK,),
        in_specs=[
            pl.BlockSpec((_BLK, _D), lambda i: (i, 0)),
            pl.BlockSpec((_NC, _BLK, _DW), lambda i: (0, i, 0)),
            pl.BlockSpec((_D, _D), lambda i: (0, 0)),
            pl.BlockSpec((1, _D), lambda i: (0, 0)),
            pl.BlockSpec((_D, _D), lambda i: (0, 0)),
            pl.BlockSpec((1, _D), lambda i: (0, 0)),
            pl.BlockSpec((1, _D), lambda i: (0, 0)),
        ],
        out_specs=[
            pl.BlockSpec((_BLK, _D), lambda i: (i, 0)),
            pl.BlockSpec((_BLK, 1), lambda i: (i, 0)),
        ],
        out_shape=[
            jax.ShapeDtypeStruct((_NP, _D), jnp.float32),
            jax.ShapeDtypeStruct((_NP, 1), jnp.float32),
        ],
    )(h, msgp, W1, b1, W2, b2, Wp)


@jax.jit
def kernel(x, edge_index, edge_type, W_in, b_in, Wr, br, W1, b1, W2, b2, Wp, bp):
    x_p = jnp.pad(x, ((0, _NP - _N), (0, 0)))
    src_p = jnp.pad(edge_index[0], (0, _EPP - _E))
    et_p = jnp.pad(edge_type, (0, _EPP - _E))
    dst_p = jnp.pad(edge_index[1], (0, _EP - _E), constant_values=_N)
    h, T0, T1, gidx2d = _table_call(x_p, W_in, b_in.reshape(1, _D), Wr,
                                    br.reshape(_R, 1, _D),
                                    src_p.reshape(_EPP // _D, _D),
                                    et_p.reshape(_EPP // _D, _D))
    t0 = T0.reshape(_R * _NP, _DW)
    t1 = T1.reshape(_R * _NP, _DW)
    gidx = gidx2d[:_EP // _D].reshape(_NS, _NJ, _CH)
    dst = dst_p.reshape(_NS, _NJ, _CH)

    zrow = jnp.zeros((_RPT, _DW), jnp.float32)
    msgp = _sc_call(t0, t1, gidx, dst, zrow)

    emb_p, pred_p = _epi_call(h, msgp, W1, b1.reshape(1, _D), W2,
                              b2.reshape(1, _D), Wp)
    return emb_p[:_N], pred_p[:_N] + bp
